# Initial kernel scaffold; baseline (speedup 1.0000x reference)
import functools

import jax
import jax.numpy as jnp
from jax import lax
from jax.experimental import pallas as pl
from jax.experimental.pallas import tpu as pltpu
from jax.experimental.pallas import tpu_sc as plsc

N_NODES = 10000
N_EDGES = 320000
D = 128
EDGE_DIM = 16

NC = 2
NS = 16
NW = NC * NS

CHUNK = 128
N_CHUNKS = N_EDGES // CHUNK
ROWS_PER_TILE = N_NODES // NS
DW = 16

_f32 = jnp.float32


def _proj_body(h_ref, ef_ref, w1a_ref, w1b_ref, w1e_ref, b1m_ref,
               a_ref, b_ref, e_ref):
    hb = h_ref[...]
    a_ref[...] = jnp.dot(hb, w1a_ref[...], preferred_element_type=_f32)
    b_ref[...] = jnp.dot(hb, w1b_ref[...], preferred_element_type=_f32) + b1m_ref[...]
    e_ref[...] = jnp.dot(ef_ref[...], w1e_ref[...], preferred_element_type=_f32)


def _project(h, ef, w1a, w1b, w1e, b1m):
    grid = 50
    nb = N_NODES // grid
    eb = N_EDGES // grid
    full = lambda i: (0, 0)
    return pl.pallas_call(
        _proj_body,
        grid=(grid,),
        in_specs=[
            pl.BlockSpec((nb, D), lambda i: (i, 0)),
            pl.BlockSpec((eb, EDGE_DIM), lambda i: (i, 0)),
            pl.BlockSpec((D, D), full),
            pl.BlockSpec((D, D), full),
            pl.BlockSpec((EDGE_DIM, D), full),
            pl.BlockSpec((1, D), full),
        ],
        out_specs=[
            pl.BlockSpec((nb, D), lambda i: (i, 0)),
            pl.BlockSpec((nb, D), lambda i: (i, 0)),
            pl.BlockSpec((eb, D), lambda i: (i, 0)),
        ],
        out_shape=[
            jax.ShapeDtypeStruct((N_NODES, D), _f32),
            jax.ShapeDtypeStruct((N_NODES, D), _f32),
            jax.ShapeDtypeStruct((N_EDGES, D), _f32),
        ],
    )(h, ef, w1a, w1b, w1e, b1m)


def _edge_body(a_hbm, b_hbm, e_hbm, src_hbm, dst_hbm,
               hp_out, dp_out,
               idx_s, idx_d, ga, gb, ge, ones_v, hacc, dacc,
               sem_a, sem_b, sem_e):
    cid = lax.axis_index("c")
    sid = lax.axis_index("s")
    wid = cid * NS + sid

    def _zero_row(r, _):
        for j in range(8):
            ga[r, pl.ds(16 * j, 16)] = jnp.zeros((16,), _f32)
        ones_v[r, pl.ds(0, 16)] = jnp.zeros((16,), _f32)
        return 0
    lax.fori_loop(0, CHUNK, _zero_row, 0)

    r0 = sid * ROWS_PER_TILE
    for j in range(5):
        dst_rows = pl.ds(r0 + j * 125, 125)
        pltpu.sync_copy(ga.at[pl.ds(0, 125)], hacc.at[dst_rows])
        pltpu.sync_copy(ones_v.at[pl.ds(0, 125)], dacc.at[dst_rows])

    def _one_row(r, _):
        ones_v[r, pl.ds(0, 16)] = jnp.ones((16,), _f32)
        return 0
    lax.fori_loop(0, CHUNK, _one_row, 0)

    plsc.subcore_barrier()

    ntrips = (N_CHUNKS // NW) + jnp.where(wid < (N_CHUNKS % NW), 1, 0)

    def _trip(t, _):
        base = (t * NW + wid) * CHUNK
        pltpu.sync_copy(src_hbm.at[pl.ds(base, CHUNK)], idx_s)
        pltpu.sync_copy(dst_hbm.at[pl.ds(base, CHUNK)], idx_d)
        cp_a = pltpu.async_copy(a_hbm.at[idx_s], ga, sem_a)
        cp_b = pltpu.async_copy(b_hbm.at[idx_d], gb, sem_b)
        cp_e = pltpu.async_copy(e_hbm.at[pl.ds(base, CHUNK)], ge, sem_e)
        cp_a.wait()
        cp_b.wait()
        cp_e.wait()

        def _row(r, _):
            for j in range(8):
                s = pl.ds(16 * j, 16)
                ga[r, s] = jnp.maximum(ga[r, s] + gb[r, s] + ge[r, s], 0.0)
            return 0
        lax.fori_loop(0, CHUNK, _row, 0)

        pltpu.sync_copy(ga, hacc.at[idx_d], add=True)
        pltpu.sync_copy(ones_v, dacc.at[idx_d], add=True)
        return 0
    lax.fori_loop(0, ntrips, _trip, 0)

    plsc.subcore_barrier()

    for j in range(5):
        rows = pl.ds(r0 + j * 125, 125)
        pltpu.sync_copy(hacc.at[rows], hp_out.at[cid, rows])
        pltpu.sync_copy(dacc.at[rows], dp_out.at[cid, rows])


@functools.partial(
    pl.kernel,
    out_type=(
        jax.ShapeDtypeStruct((NC, N_NODES, D), _f32),
        jax.ShapeDtypeStruct((NC, N_NODES, DW), _f32),
    ),
    mesh=plsc.VectorSubcoreMesh(core_axis_name="c", subcore_axis_name="s"),
    scratch_types=[
        pltpu.VMEM((CHUNK,), jnp.int32),
        pltpu.VMEM((CHUNK,), jnp.int32),
        pltpu.VMEM((CHUNK, D), _f32),
        pltpu.VMEM((CHUNK, D), _f32),
        pltpu.VMEM((CHUNK, D), _f32),
        pltpu.VMEM((CHUNK, DW), _f32),
        pltpu.VMEM_SHARED((N_NODES, D), _f32),
        pltpu.VMEM_SHARED((N_NODES, DW), _f32),
        pltpu.SemaphoreType.DMA,
        pltpu.SemaphoreType.DMA,
        pltpu.SemaphoreType.DMA,
    ],
)
def _edge_scatter(a_hbm, b_hbm, e_hbm, src_hbm, dst_hbm, hp_out, dp_out,
                  idx_s, idx_d, ga, gb, ge, ones_v, hacc, dacc,
                  sem_a, sem_b, sem_e):
    _edge_body(a_hbm, b_hbm, e_hbm, src_hbm, dst_hbm, hp_out, dp_out,
               idx_s, idx_d, ga, gb, ge, ones_v, hacc, dacc,
               sem_a, sem_b, sem_e)


def _update_body(hp_ref, dp_ref, h_ref, w2m_ref, b2m_ref,
                 w1uh_ref, w1ua_ref, b1u_ref, w2u_ref, b2u_ref,
                 gamma_ref, beta_ref, o_ref):
    hs = hp_ref[0] + hp_ref[1]
    deg = dp_ref[0, :, 0:1] + dp_ref[1, :, 0:1]
    num = jnp.dot(hs, w2m_ref[...], preferred_element_type=_f32) + deg * b2m_ref[...]
    agg = num / jnp.maximum(deg, 1.0)
    u = jnp.dot(h_ref[...], w1uh_ref[...], preferred_element_type=_f32)
    u = u + jnp.dot(agg, w1ua_ref[...], preferred_element_type=_f32) + b1u_ref[...]
    u = jnp.maximum(u, 0.0)
    u = jnp.dot(u, w2u_ref[...], preferred_element_type=_f32) + b2u_ref[...]
    mu = jnp.mean(u, axis=1, keepdims=True)
    var = jnp.mean((u - mu) * (u - mu), axis=1, keepdims=True)
    o_ref[...] = (u - mu) * lax.rsqrt(var + 1e-5) * gamma_ref[...] + beta_ref[...]


def _update(hp, dp, h, w2m, b2m, w1uh, w1ua, b1u, w2u, b2u, gamma, beta):
    grid = 5
    nb = N_NODES // grid
    full = lambda i: (0, 0)
    full3 = lambda i: (0, i, 0)
    return pl.pallas_call(
        _update_body,
        grid=(grid,),
        in_specs=[
            pl.BlockSpec((NC, nb, D), full3),
            pl.BlockSpec((NC, nb, DW), full3),
            pl.BlockSpec((nb, D), lambda i: (i, 0)),
            pl.BlockSpec((D, D), full),
            pl.BlockSpec((1, D), full),
            pl.BlockSpec((D, D), full),
            pl.BlockSpec((D, D), full),
            pl.BlockSpec((1, D), full),
            pl.BlockSpec((D, D), full),
            pl.BlockSpec((1, D), full),
            pl.BlockSpec((1, D), full),
            pl.BlockSpec((1, D), full),
        ],
        out_specs=pl.BlockSpec((nb, D), lambda i: (i, 0)),
        out_shape=jax.ShapeDtypeStruct((N_NODES, D), _f32),
    )(hp, dp, h, w2m, b2m, w1uh, w1ua, b1u, w2u, b2u, gamma, beta)


def kernel(h, edge_index, edge_features, W1m, b1m, W2m, b2m,
           W1u, b1u, W2u, b2u, gamma, beta):
    w1a = W1m[:D]
    w1b = W1m[D:2 * D]
    w1e = W1m[2 * D:]
    src = edge_index[0]
    dst = edge_index[1]
    row = lambda v: v.reshape(1, D)

    a, b, e = _project(h, edge_features, w1a, w1b, w1e, row(b1m))
    hp, dp = _edge_scatter(a, b, e, src, dst)
    return _update(hp, dp, h, W2m, row(b2m), W1u[:D], W1u[D:], row(b1u),
                   W2u, row(b2u), row(gamma), row(beta))


# speedup vs baseline: 3.6176x; 3.6176x over previous
import functools

import jax
import jax.numpy as jnp
from jax import lax
from jax.experimental import pallas as pl
from jax.experimental.pallas import tpu as pltpu
from jax.experimental.pallas import tpu_sc as plsc

N_NODES = 10000
N_EDGES = 320000
D = 128
EDGE_DIM = 16

NC = 2
NS = 16
NW = NC * NS

CHUNK = 64
N_CHUNKS = N_EDGES // CHUNK
NP = 10240
ROWS_PER_TILE = NP // NS
DW = 16
DT = D + DW

_f32 = jnp.float32


def _proj_body(h_ref, ef_ref, w1a_ref, w1b_ref, w1e_ref, b1m_ref,
               a_ref, b_ref, e_ref):
    hb = h_ref[...]
    a_ref[...] = jnp.dot(hb, w1a_ref[...], preferred_element_type=_f32)
    b_ref[...] = jnp.dot(hb, w1b_ref[...], preferred_element_type=_f32) + b1m_ref[...]
    e_ref[...] = jnp.dot(ef_ref[...], w1e_ref[...], preferred_element_type=_f32)


def _project(h, ef, w1a, w1b, w1e, b1m):
    grid = 50
    nb = N_NODES // grid
    eb = N_EDGES // grid
    full = lambda i: (0, 0)
    return pl.pallas_call(
        _proj_body,
        grid=(grid,),
        in_specs=[
            pl.BlockSpec((nb, D), lambda i: (i, 0)),
            pl.BlockSpec((eb, EDGE_DIM), lambda i: (i, 0)),
            pl.BlockSpec((D, D), full),
            pl.BlockSpec((D, D), full),
            pl.BlockSpec((EDGE_DIM, D), full),
            pl.BlockSpec((1, D), full),
        ],
        out_specs=[
            pl.BlockSpec((nb, D), lambda i: (i, 0)),
            pl.BlockSpec((nb, D), lambda i: (i, 0)),
            pl.BlockSpec((eb, D), lambda i: (i, 0)),
        ],
        out_shape=[
            jax.ShapeDtypeStruct((N_NODES, D), _f32),
            jax.ShapeDtypeStruct((N_NODES, D), _f32),
            jax.ShapeDtypeStruct((N_EDGES, D), _f32),
        ],
    )(h, ef, w1a, w1b, w1e, b1m)


def _edge_body(zh_hbm, a_hbm, b_hbm, e_hbm, src_hbm, dst_hbm,
               hp_out, dp_out,
               idx_s, idx_d, ga, gb, ge, degloc, hacc,
               sem_a, sem_b, sem_e):
    cid = lax.axis_index("c")
    sid = lax.axis_index("s")
    wid = cid * NS + sid

    r0 = sid * ROWS_PER_TILE
    pltpu.sync_copy(zh_hbm, hacc.at[pl.ds(r0, ROWS_PER_TILE)])

    def _zero_deg(i, _):
        degloc[pl.ds(i * 16, 16)] = jnp.zeros((16,), _f32)
        return 0
    lax.fori_loop(0, NP // 16, _zero_deg, 0)

    plsc.subcore_barrier()

    ntrips = (N_CHUNKS // NW) + jnp.where(wid < (N_CHUNKS % NW), 1, 0)
    ones16 = jnp.ones((16,), _f32)

    def _trip(t, _):
        base = (t * NW + wid) * CHUNK
        pltpu.sync_copy(src_hbm.at[pl.ds(base, CHUNK)], idx_s)
        pltpu.sync_copy(dst_hbm.at[pl.ds(base, CHUNK)], idx_d)
        cp_a = pltpu.async_copy(a_hbm.at[idx_s], ga, sem_a)
        cp_b = pltpu.async_copy(b_hbm.at[idx_d], gb, sem_b)
        cp_e = pltpu.async_copy(e_hbm.at[pl.ds(base, CHUNK)], ge, sem_e)
        cp_a.wait()
        cp_b.wait()
        cp_e.wait()

        def _row(r, _):
            for j in range(8):
                s = pl.ds(16 * j, 16)
                ga[r, s] = jnp.maximum(ga[r, s] + gb[r, s] + ge[r, s], 0.0)
            return 0
        lax.fori_loop(0, CHUNK, _row, 0)

        for j in range(CHUNK // 16):
            iv = idx_d[pl.ds(16 * j, 16)]
            plsc.addupdate_scatter(degloc, [iv], ones16)

        pltpu.sync_copy(ga, hacc.at[idx_d], add=True)
        return 0
    lax.fori_loop(0, ntrips, _trip, 0)

    plsc.subcore_barrier()

    for j in range(ROWS_PER_TILE // CHUNK):
        rows = pl.ds(r0 + j * CHUNK, CHUNK)
        out_rows = pl.ds(cid * NP + r0 + j * CHUNK, CHUNK)
        pltpu.sync_copy(hacc.at[rows], ga)
        pltpu.sync_copy(ga, hp_out.at[out_rows])
    pltpu.sync_copy(degloc, dp_out.at[pl.ds(wid * NP, NP)])


@functools.partial(
    pl.kernel,
    out_type=(
        jax.ShapeDtypeStruct((NC * NP, D), _f32),
        jax.ShapeDtypeStruct((NW * NP,), _f32),
    ),
    mesh=plsc.VectorSubcoreMesh(core_axis_name="c", subcore_axis_name="s"),
    compiler_params=pltpu.CompilerParams(needs_layout_passes=False),
    scratch_types=[
        pltpu.VMEM((CHUNK,), jnp.int32),
        pltpu.VMEM((CHUNK,), jnp.int32),
        pltpu.VMEM((CHUNK, D), _f32),
        pltpu.VMEM((CHUNK, D), _f32),
        pltpu.VMEM((CHUNK, D), _f32),
        pltpu.VMEM((NP,), _f32),
        pltpu.VMEM_SHARED((NP, D), _f32),
        pltpu.SemaphoreType.DMA,
        pltpu.SemaphoreType.DMA,
        pltpu.SemaphoreType.DMA,
    ],
)
def _edge_scatter(zh_hbm, a_hbm, b_hbm, e_hbm, src_hbm, dst_hbm,
                  hp_out, dp_out,
                  idx_s, idx_d, ga, gb, ge, degloc, hacc,
                  sem_a, sem_b, sem_e):
    _edge_body(zh_hbm, a_hbm, b_hbm, e_hbm, src_hbm, dst_hbm,
               hp_out, dp_out,
               idx_s, idx_d, ga, gb, ge, degloc, hacc,
               sem_a, sem_b, sem_e)


def _update_body(hp_ref, dp_ref, h_ref, w2m_ref, b2m_ref,
                 w1uh_ref, w1ua_ref, b1u_ref, w2u_ref, b2u_ref,
                 gamma_ref, beta_ref, o_ref):
    hs = hp_ref[0] + hp_ref[1]
    deg = jnp.sum(dp_ref[...], axis=0)[:, None]
    num = jnp.dot(hs, w2m_ref[...], preferred_element_type=_f32) + deg * b2m_ref[...]
    agg = num / jnp.maximum(deg, 1.0)
    u = jnp.dot(h_ref[...], w1uh_ref[...], preferred_element_type=_f32)
    u = u + jnp.dot(agg, w1ua_ref[...], preferred_element_type=_f32) + b1u_ref[...]
    u = jnp.maximum(u, 0.0)
    u = jnp.dot(u, w2u_ref[...], preferred_element_type=_f32) + b2u_ref[...]
    mu = jnp.mean(u, axis=1, keepdims=True)
    var = jnp.mean((u - mu) * (u - mu), axis=1, keepdims=True)
    o_ref[...] = (u - mu) * lax.rsqrt(var + 1e-5) * gamma_ref[...] + beta_ref[...]


def _update(hp, dp, h, w2m, b2m, w1uh, w1ua, b1u, w2u, b2u, gamma, beta):
    grid = 5
    nb = 2048
    full = lambda i: (0, 0)
    return pl.pallas_call(
        _update_body,
        grid=(grid,),
        in_specs=[
            pl.BlockSpec((NC, nb, D), lambda i: (0, i, 0)),
            pl.BlockSpec((NW, nb), lambda i: (0, i)),
            pl.BlockSpec((nb, D), lambda i: (i, 0)),
            pl.BlockSpec((D, D), full),
            pl.BlockSpec((1, D), full),
            pl.BlockSpec((D, D), full),
            pl.BlockSpec((D, D), full),
            pl.BlockSpec((1, D), full),
            pl.BlockSpec((D, D), full),
            pl.BlockSpec((1, D), full),
            pl.BlockSpec((1, D), full),
            pl.BlockSpec((1, D), full),
        ],
        out_specs=pl.BlockSpec((nb, D), lambda i: (i, 0)),
        out_shape=jax.ShapeDtypeStruct((N_NODES, D), _f32),
    )(hp, dp, h, w2m, b2m, w1uh, w1ua, b1u, w2u, b2u, gamma, beta)


def kernel(h, edge_index, edge_features, W1m, b1m, W2m, b2m,
           W1u, b1u, W2u, b2u, gamma, beta):
    w1a = W1m[:D]
    w1b = W1m[D:2 * D]
    w1e = W1m[2 * D:]
    src = edge_index[0]
    dst = edge_index[1]
    row = lambda v: v.reshape(1, D)

    a, b, e = _project(h, edge_features, w1a, w1b, w1e, row(b1m))
    zh = jnp.zeros((ROWS_PER_TILE, D), _f32)
    hp, dp = _edge_scatter(zh, a, b, e, src, dst)
    hp = hp.reshape(NC, NP, D)
    dp = dp.reshape(NW, NP)
    return _update(hp, dp, h, W2m, row(b2m), W1u[:D], W1u[D:], row(b1u),
                   W2u, row(b2u), row(gamma), row(beta))


# speedup vs baseline: 4.4262x; 1.2235x over previous
import functools

import jax
import jax.numpy as jnp
from jax import lax
from jax.experimental import pallas as pl
from jax.experimental.pallas import tpu as pltpu
from jax.experimental.pallas import tpu_sc as plsc

N_NODES = 10000
N_EDGES = 320000
D = 128
EDGE_DIM = 16

NC = 2
NS = 16
NW = NC * NS

CHUNK = 40
N_CHUNKS = N_EDGES // CHUNK
TRIPS = N_CHUNKS // 32
NP = 10240
ROWS_PER_TILE = NP // NS
DW = 16
DT = D + DW

_f32 = jnp.float32


def _proj_body(h_ref, ef_ref, w1a_ref, w1b_ref, w1e_ref, b1m_ref,
               a_ref, b_ref, e_ref):
    hb = h_ref[...]
    a_ref[...] = jnp.dot(hb, w1a_ref[...], preferred_element_type=_f32)
    b_ref[...] = jnp.dot(hb, w1b_ref[...], preferred_element_type=_f32) + b1m_ref[...]
    e_ref[...] = jnp.dot(ef_ref[...], w1e_ref[...], preferred_element_type=_f32)


def _project(h, ef, w1a, w1b, w1e, b1m):
    grid = 50
    nb = N_NODES // grid
    eb = N_EDGES // grid
    full = lambda i: (0, 0)
    return pl.pallas_call(
        _proj_body,
        grid=(grid,),
        in_specs=[
            pl.BlockSpec((nb, D), lambda i: (i, 0)),
            pl.BlockSpec((eb, EDGE_DIM), lambda i: (i, 0)),
            pl.BlockSpec((D, D), full),
            pl.BlockSpec((D, D), full),
            pl.BlockSpec((EDGE_DIM, D), full),
            pl.BlockSpec((1, D), full),
        ],
        out_specs=[
            pl.BlockSpec((nb, D), lambda i: (i, 0)),
            pl.BlockSpec((nb, D), lambda i: (i, 0)),
            pl.BlockSpec((eb, D), lambda i: (i, 0)),
        ],
        out_shape=[
            jax.ShapeDtypeStruct((N_NODES, D), _f32),
            jax.ShapeDtypeStruct((N_NODES, D), _f32),
            jax.ShapeDtypeStruct((N_EDGES, D), _f32),
        ],
    )(h, ef, w1a, w1b, w1e, b1m)


def _edge_body(zh_hbm, a_hbm, b_hbm, e_hbm, src_hbm, dst_hbm,
               hp_out, dp_out,
               idx_s0, idx_d0, ga0, gb0, ge0,
               idx_s1, idx_d1, ga1, gb1, ge1,
               degloc, hacc,
               sa0, sb0, se0, sa1, sb1, se1):
    cid = lax.axis_index("c")
    sid = lax.axis_index("s")
    wid = cid * NS + sid

    idx_s = [idx_s0, idx_s1]
    idx_d = [idx_d0, idx_d1]
    ga = [ga0, ga1]
    gb = [gb0, gb1]
    ge = [ge0, ge1]
    sa = [sa0, sa1]
    sb = [sb0, sb1]
    se = [se0, se1]

    r0 = sid * ROWS_PER_TILE
    pltpu.sync_copy(zh_hbm, hacc.at[pl.ds(r0, ROWS_PER_TILE)])

    def _zero_deg(i, _):
        degloc[pl.ds(i * 16, 16)] = jnp.zeros((16,), _f32)
        return 0
    lax.fori_loop(0, NP // 16, _zero_deg, 0)

    plsc.subcore_barrier()

    ones16 = jnp.ones((16,), _f32)
    tail_mask = lax.iota(jnp.int32, 16) >= (2 * 16 - (CHUNK - 16))

    def _prefetch(t, k):
        base = (t * NW + wid) * CHUNK
        pltpu.sync_copy(src_hbm.at[pl.ds(base, CHUNK)], idx_s[k])
        pltpu.sync_copy(dst_hbm.at[pl.ds(base, CHUNK)], idx_d[k])
        pltpu.async_copy(a_hbm.at[idx_s[k]], ga[k], sa[k])
        pltpu.async_copy(b_hbm.at[idx_d[k]], gb[k], sb[k])
        pltpu.async_copy(e_hbm.at[pl.ds(base, CHUNK)], ge[k], se[k])

    def _process(t, k):
        pltpu.make_async_copy(a_hbm.at[idx_s[k]], ga[k], sa[k]).wait()
        pltpu.make_async_copy(b_hbm.at[idx_d[k]], gb[k], sb[k]).wait()
        pltpu.make_async_copy(e_hbm.at[pl.ds(0, CHUNK)], ge[k], se[k]).wait()

        def _row(r, _):
            for j in range(8):
                s = pl.ds(16 * j, 16)
                ga[k][r, s] = jnp.maximum(ga[k][r, s] + gb[k][r, s] + ge[k][r, s], 0.0)
            return 0
        lax.fori_loop(0, CHUNK, _row, 0)

        plsc.addupdate_scatter(degloc, [idx_d[k][pl.ds(0, 16)]], ones16)
        plsc.addupdate_scatter(degloc, [idx_d[k][pl.ds(16, 16)]], ones16)
        plsc.addupdate_scatter(
            degloc, [idx_d[k][pl.ds(CHUNK - 16, 16)]], ones16, mask=tail_mask)

        pltpu.sync_copy(ga[k], hacc.at[idx_d[k]], add=True)

    _prefetch(0, 0)

    def _pair(i, _):
        t0 = 2 * i
        _prefetch(t0 + 1, 1)
        _process(t0, 0)

        @pl.when(t0 + 2 < TRIPS)
        def _():
            _prefetch(t0 + 2, 0)
        _process(t0 + 1, 1)
        return 0
    lax.fori_loop(0, TRIPS // 2, _pair, 0)

    plsc.subcore_barrier()

    for j in range(ROWS_PER_TILE // CHUNK):
        rows = pl.ds(r0 + j * CHUNK, CHUNK)
        out_rows = pl.ds(cid * NP + r0 + j * CHUNK, CHUNK)
        pltpu.sync_copy(hacc.at[rows], ga0)
        pltpu.sync_copy(ga0, hp_out.at[out_rows])
    pltpu.sync_copy(degloc, dp_out.at[pl.ds(wid * NP, NP)])


@functools.partial(
    pl.kernel,
    out_type=(
        jax.ShapeDtypeStruct((NC * NP, D), _f32),
        jax.ShapeDtypeStruct((NW * NP,), _f32),
    ),
    mesh=plsc.VectorSubcoreMesh(core_axis_name="c", subcore_axis_name="s"),
    compiler_params=pltpu.CompilerParams(needs_layout_passes=False),
    scratch_types=[
        pltpu.VMEM((CHUNK,), jnp.int32),
        pltpu.VMEM((CHUNK,), jnp.int32),
        pltpu.VMEM((CHUNK, D), _f32),
        pltpu.VMEM((CHUNK, D), _f32),
        pltpu.VMEM((CHUNK, D), _f32),
        pltpu.VMEM((CHUNK,), jnp.int32),
        pltpu.VMEM((CHUNK,), jnp.int32),
        pltpu.VMEM((CHUNK, D), _f32),
        pltpu.VMEM((CHUNK, D), _f32),
        pltpu.VMEM((CHUNK, D), _f32),
        pltpu.VMEM((NP,), _f32),
        pltpu.VMEM_SHARED((NP, D), _f32),
        pltpu.SemaphoreType.DMA,
        pltpu.SemaphoreType.DMA,
        pltpu.SemaphoreType.DMA,
        pltpu.SemaphoreType.DMA,
        pltpu.SemaphoreType.DMA,
        pltpu.SemaphoreType.DMA,
    ],
)
def _edge_scatter(zh_hbm, a_hbm, b_hbm, e_hbm, src_hbm, dst_hbm,
                  hp_out, dp_out,
                  idx_s0, idx_d0, ga0, gb0, ge0,
                  idx_s1, idx_d1, ga1, gb1, ge1,
                  degloc, hacc,
                  sa0, sb0, se0, sa1, sb1, se1):
    _edge_body(zh_hbm, a_hbm, b_hbm, e_hbm, src_hbm, dst_hbm,
               hp_out, dp_out,
               idx_s0, idx_d0, ga0, gb0, ge0,
               idx_s1, idx_d1, ga1, gb1, ge1,
               degloc, hacc,
               sa0, sb0, se0, sa1, sb1, se1)


def _update_body(hp_ref, dp_ref, h_ref, w2m_ref, b2m_ref,
                 w1uh_ref, w1ua_ref, b1u_ref, w2u_ref, b2u_ref,
                 gamma_ref, beta_ref, o_ref):
    hs = hp_ref[0] + hp_ref[1]
    deg = jnp.sum(dp_ref[...], axis=0)[:, None]
    num = jnp.dot(hs, w2m_ref[...], preferred_element_type=_f32) + deg * b2m_ref[...]
    agg = num / jnp.maximum(deg, 1.0)
    u = jnp.dot(h_ref[...], w1uh_ref[...], preferred_element_type=_f32)
    u = u + jnp.dot(agg, w1ua_ref[...], preferred_element_type=_f32) + b1u_ref[...]
    u = jnp.maximum(u, 0.0)
    u = jnp.dot(u, w2u_ref[...], preferred_element_type=_f32) + b2u_ref[...]
    mu = jnp.mean(u, axis=1, keepdims=True)
    var = jnp.mean((u - mu) * (u - mu), axis=1, keepdims=True)
    o_ref[...] = (u - mu) * lax.rsqrt(var + 1e-5) * gamma_ref[...] + beta_ref[...]


def _update(hp, dp, h, w2m, b2m, w1uh, w1ua, b1u, w2u, b2u, gamma, beta):
    grid = 5
    nb = 2048
    full = lambda i: (0, 0)
    return pl.pallas_call(
        _update_body,
        grid=(grid,),
        in_specs=[
            pl.BlockSpec((NC, nb, D), lambda i: (0, i, 0)),
            pl.BlockSpec((NW, nb), lambda i: (0, i)),
            pl.BlockSpec((nb, D), lambda i: (i, 0)),
            pl.BlockSpec((D, D), full),
            pl.BlockSpec((1, D), full),
            pl.BlockSpec((D, D), full),
            pl.BlockSpec((D, D), full),
            pl.BlockSpec((1, D), full),
            pl.BlockSpec((D, D), full),
            pl.BlockSpec((1, D), full),
            pl.BlockSpec((1, D), full),
            pl.BlockSpec((1, D), full),
        ],
        out_specs=pl.BlockSpec((nb, D), lambda i: (i, 0)),
        out_shape=jax.ShapeDtypeStruct((N_NODES, D), _f32),
    )(hp, dp, h, w2m, b2m, w1uh, w1ua, b1u, w2u, b2u, gamma, beta)


def kernel(h, edge_index, edge_features, W1m, b1m, W2m, b2m,
           W1u, b1u, W2u, b2u, gamma, beta):
    w1a = W1m[:D]
    w1b = W1m[D:2 * D]
    w1e = W1m[2 * D:]
    src = edge_index[0]
    dst = edge_index[1]
    row = lambda v: v.reshape(1, D)

    a, b, e = _project(h, edge_features, w1a, w1b, w1e, row(b1m))
    zh = jnp.zeros((ROWS_PER_TILE, D), _f32)
    hp, dp = _edge_scatter(zh, a, b, e, src, dst)
    hp = hp.reshape(NC, NP, D)
    dp = dp.reshape(NW, NP)
    return _update(hp, dp, h, W2m, row(b2m), W1u[:D], W1u[D:], row(b1u),
                   W2u, row(b2u), row(gamma), row(beta))
